# baseline (speedup 1.0000x reference)
import functools

import jax
import jax.numpy as jnp
from jax import lax
from jax.experimental import pallas as pl
from jax.experimental.pallas import tpu as pltpu
from jax.experimental.pallas import tpu_sc as plsc

N_NODES = 10000
N_EDGES = 320000
NFEAT = 128
NCLASS = 40
DPAD = 64

NC = 2
NS = 16
CHUNK = 128
N_CHUNKS = N_EDGES // CHUNK
CPT = N_CHUNKS // (NC * NS)
TAIL = N_CHUNKS - CPT * NC * NS
GRP = 26
ROWS_PER_TILE = N_NODES // NS


def _make_spmm(D):
  mesh = plsc.VectorSubcoreMesh(core_axis_name="c", subcore_axis_name="s")
  params = pltpu.CompilerParams(use_tc_tiling_on_sc=False)

  @functools.partial(
      pl.kernel,
      out_type=jax.ShapeDtypeStruct((N_NODES, NC * D), jnp.float32),
      compiler_params=params,
      mesh=mesh,
      scratch_types=[
          pltpu.VMEM((GRP, CHUNK), jnp.int32),
          pltpu.VMEM((GRP, CHUNK), jnp.int32),
          pltpu.VMEM((CHUNK, D), jnp.float32),
          pltpu.VMEM((CHUNK, D), jnp.float32),
          pltpu.VMEM_SHARED((N_NODES, D), jnp.float32),
          pltpu.SemaphoreType.DMA,
          pltpu.SemaphoreType.DMA,
      ],
  )
  def spmm(table_hbm, ei_hbm, out_hbm, sidx, didx, r0, r1, acc, sem0, sem1):
    c = lax.axis_index("c")
    s = lax.axis_index("s")
    wid = c * NS + s

    zero16 = jnp.zeros((16,), jnp.float32)

    def zrow(i, carry):
      for j in range(D // 16):
        r0[i, pl.ds(j * 16, 16)] = zero16
      return carry

    lax.fori_loop(0, CHUNK, zrow, 0)
    for k in range(4):
      pltpu.sync_copy(r0, acc.at[pl.ds(s * ROWS_PER_TILE + k * 128, 128)])
    pltpu.sync_copy(
        r0.at[pl.ds(0, ROWS_PER_TILE - 4 * 128)],
        acc.at[pl.ds(s * ROWS_PER_TILE + 4 * 128, ROWS_PER_TILE - 4 * 128)])

    def wait0():
      pltpu.make_async_copy(table_hbm.at[sidx.at[0]], r0, sem0).wait()

    def wait1():
      pltpu.make_async_copy(table_hbm.at[sidx.at[0]], r1, sem1).wait()

    def run_group(base, n_pairs, barrier_after_prime=False):
      pltpu.sync_copy(ei_hbm.at[0, pl.ds(base, 2 * n_pairs)],
                      sidx.at[pl.ds(0, 2 * n_pairs)])
      pltpu.sync_copy(ei_hbm.at[1, pl.ds(base, 2 * n_pairs)],
                      didx.at[pl.ds(0, 2 * n_pairs)])
      pltpu.async_copy(table_hbm.at[sidx.at[0]], r0, sem0)
      pltpu.async_copy(table_hbm.at[sidx.at[1]], r1, sem1)
      if barrier_after_prime:
        plsc.subcore_barrier()

      last = 2 * n_pairs - 1

      def body(i, carry):
        j0 = 2 * i
        wait0()
        pltpu.sync_copy(r0, acc.at[didx.at[j0]], add=True)
        pltpu.async_copy(
            table_hbm.at[sidx.at[jnp.minimum(j0 + 2, last)]], r0, sem0)
        wait1()
        pltpu.sync_copy(r1, acc.at[didx.at[j0 + 1]], add=True)
        pltpu.async_copy(
            table_hbm.at[sidx.at[jnp.minimum(j0 + 3, last)]], r1, sem1)
        return carry

      lax.fori_loop(0, n_pairs, body, 0)
      wait0()
      wait1()

    for t in range(CPT // GRP):
      run_group(wid * CPT + t * GRP, GRP // 2, barrier_after_prime=(t == 0))

    @pl.when(wid == 0)
    def _():
      run_group(NC * NS * CPT, TAIL // 2)

    plsc.subcore_barrier()

    pltpu.sync_copy(
        acc.at[pl.ds(s * ROWS_PER_TILE, ROWS_PER_TILE)],
        out_hbm.at[pl.ds(s * ROWS_PER_TILE, ROWS_PER_TILE),
                   pl.ds(c * D, D)])

  return spmm


_spmm128 = _make_spmm(NFEAT)
_spmm64 = _make_spmm(DPAD)


_BM = 2000
_NB = N_NODES // _BM


def _mid_body(p_ref, w1_ref, w2_ref, g_ref):
  x = p_ref[:, :NFEAT] + p_ref[:, NFEAT:]
  h = jnp.maximum(
      jnp.dot(x, w1_ref[...], preferred_element_type=jnp.float32), 0.0)
  g_ref[...] = jnp.dot(h, w2_ref[...], preferred_element_type=jnp.float32)


def _mid(p, W1, W2p):
  return pl.pallas_call(
      _mid_body,
      grid=(_NB,),
      in_specs=[
          pl.BlockSpec((_BM, NC * NFEAT), lambda i: (i, 0)),
          pl.BlockSpec((NFEAT, NFEAT), lambda i: (0, 0)),
          pl.BlockSpec((NFEAT, DPAD), lambda i: (0, 0)),
      ],
      out_specs=pl.BlockSpec((_BM, DPAD), lambda i: (i, 0)),
      out_shape=jax.ShapeDtypeStruct((N_NODES, DPAD), jnp.float32),
  )(p, W1, W2p)


def _fin_body(r_ref, b2_ref, o_ref):
  y = r_ref[:, :DPAD] + r_ref[:, DPAD:] + b2_ref[...]
  col = lax.broadcasted_iota(jnp.int32, y.shape, 1)
  ym = jnp.where(col < NCLASS, y, -jnp.inf)
  m = jnp.max(ym, axis=1, keepdims=True)
  lse = jnp.log(jnp.sum(jnp.exp(ym - m), axis=1, keepdims=True)) + m
  o_ref[...] = (y - lse)[:, :NCLASS]


def _fin(r, b2p):
  return pl.pallas_call(
      _fin_body,
      grid=(_NB,),
      in_specs=[
          pl.BlockSpec((_BM, NC * DPAD), lambda i: (i, 0)),
          pl.BlockSpec((1, DPAD), lambda i: (0, 0)),
      ],
      out_specs=pl.BlockSpec((_BM, NCLASS), lambda i: (i, 0)),
      out_shape=jax.ShapeDtypeStruct((N_NODES, NCLASS), jnp.float32),
  )(r, b2p)


def kernel(features, edge_index, W1, W2, b2):
  ei = edge_index.reshape(2, N_CHUNKS, CHUNK)
  p = _spmm128(features, ei)
  W2p = jnp.pad(W2, ((0, 0), (0, DPAD - NCLASS)))
  g = _mid(p, W1, W2p)
  r = _spmm64(g, ei)
  b2p = jnp.pad(b2, (0, DPAD - NCLASS)).reshape(1, DPAD)
  return _fin(r, b2p)


# speedup vs baseline: 1.0290x; 1.0290x over previous
import functools

import jax
import jax.numpy as jnp
from jax import lax
from jax.experimental import pallas as pl
from jax.experimental.pallas import tpu as pltpu
from jax.experimental.pallas import tpu_sc as plsc

N_NODES = 10000
N_EDGES = 320000
NFEAT = 128
NCLASS = 40
DPAD = 64

NC = 2
NS = 16
CHUNK = 128
N_CHUNKS = N_EDGES // CHUNK
CPT = N_CHUNKS // (NC * NS)
TAIL = N_CHUNKS - CPT * NC * NS
GRP = 26
ROWS_PER_TILE = N_NODES // NS


def _make_spmm(D):
  mesh = plsc.VectorSubcoreMesh(core_axis_name="c", subcore_axis_name="s")
  params = pltpu.CompilerParams(use_tc_tiling_on_sc=False)

  @functools.partial(
      pl.kernel,
      out_type=jax.ShapeDtypeStruct((NC * N_NODES, D), jnp.float32),
      compiler_params=params,
      mesh=mesh,
      scratch_types=[
          pltpu.VMEM((GRP, CHUNK), jnp.int32),
          pltpu.VMEM((GRP, CHUNK), jnp.int32),
          pltpu.VMEM((CHUNK, D), jnp.float32),
          pltpu.VMEM((CHUNK, D), jnp.float32),
          pltpu.VMEM_SHARED((N_NODES, D), jnp.float32),
          pltpu.SemaphoreType.DMA,
          pltpu.SemaphoreType.DMA,
      ],
  )
  def spmm(table_hbm, ei_hbm, out_hbm, sidx, didx, r0, r1, acc, sem0, sem1):
    c = lax.axis_index("c")
    s = lax.axis_index("s")
    wid = c * NS + s

    zero16 = jnp.zeros((16,), jnp.float32)

    def zrow(i, carry):
      for j in range(D // 16):
        r0[i, pl.ds(j * 16, 16)] = zero16
      return carry

    lax.fori_loop(0, CHUNK, zrow, 0)
    for k in range(4):
      pltpu.sync_copy(r0, acc.at[pl.ds(s * ROWS_PER_TILE + k * 128, 128)])
    pltpu.sync_copy(
        r0.at[pl.ds(0, ROWS_PER_TILE - 4 * 128)],
        acc.at[pl.ds(s * ROWS_PER_TILE + 4 * 128, ROWS_PER_TILE - 4 * 128)])

    def wait0():
      pltpu.make_async_copy(table_hbm.at[sidx.at[0]], r0, sem0).wait()

    def wait1():
      pltpu.make_async_copy(table_hbm.at[sidx.at[0]], r1, sem1).wait()

    def run_group(base, n_pairs, barrier_after_prime=False):
      pltpu.sync_copy(ei_hbm.at[0, pl.ds(base, 2 * n_pairs)],
                      sidx.at[pl.ds(0, 2 * n_pairs)])
      pltpu.sync_copy(ei_hbm.at[1, pl.ds(base, 2 * n_pairs)],
                      didx.at[pl.ds(0, 2 * n_pairs)])
      pltpu.async_copy(table_hbm.at[sidx.at[0]], r0, sem0)
      pltpu.async_copy(table_hbm.at[sidx.at[1]], r1, sem1)
      if barrier_after_prime:
        plsc.subcore_barrier()

      last = 2 * n_pairs - 1

      def body(i, carry):
        j0 = 2 * i
        wait0()
        pltpu.sync_copy(r0, acc.at[didx.at[j0]], add=True)
        pltpu.async_copy(
            table_hbm.at[sidx.at[jnp.minimum(j0 + 2, last)]], r0, sem0)
        wait1()
        pltpu.sync_copy(r1, acc.at[didx.at[j0 + 1]], add=True)
        pltpu.async_copy(
            table_hbm.at[sidx.at[jnp.minimum(j0 + 3, last)]], r1, sem1)
        return carry

      lax.fori_loop(0, n_pairs, body, 0)
      wait0()
      wait1()

    for t in range(CPT // GRP):
      run_group(wid * CPT + t * GRP, GRP // 2, barrier_after_prime=(t == 0))

    @pl.when(wid < TAIL)
    def _():
      base = NC * NS * CPT + wid
      pltpu.sync_copy(ei_hbm.at[0, pl.ds(base, 1)], sidx.at[pl.ds(0, 1)])
      pltpu.sync_copy(ei_hbm.at[1, pl.ds(base, 1)], didx.at[pl.ds(0, 1)])
      pltpu.async_copy(table_hbm.at[sidx.at[0]], r0, sem0)
      wait0()
      pltpu.sync_copy(r0, acc.at[didx.at[0]], add=True)

    plsc.subcore_barrier()

    pltpu.sync_copy(
        acc.at[pl.ds(s * ROWS_PER_TILE, ROWS_PER_TILE)],
        out_hbm.at[pl.ds(c * N_NODES + s * ROWS_PER_TILE, ROWS_PER_TILE)])

  return spmm


_spmm128 = _make_spmm(NFEAT)
_spmm64 = _make_spmm(DPAD)


_BM = 2000
_NB = N_NODES // _BM


def _mid_body(q0_ref, q1_ref, w1_ref, w2_ref, g_ref):
  x = q0_ref[...] + q1_ref[...]
  h = jnp.maximum(
      jnp.dot(x, w1_ref[...], preferred_element_type=jnp.float32), 0.0)
  g_ref[...] = jnp.dot(h, w2_ref[...], preferred_element_type=jnp.float32)


def _mid(p, W1, W2p):
  return pl.pallas_call(
      _mid_body,
      grid=(_NB,),
      in_specs=[
          pl.BlockSpec((_BM, NFEAT), lambda i: (i, 0)),
          pl.BlockSpec((_BM, NFEAT), lambda i: (i + _NB, 0)),
          pl.BlockSpec((NFEAT, NFEAT), lambda i: (0, 0)),
          pl.BlockSpec((NFEAT, DPAD), lambda i: (0, 0)),
      ],
      out_specs=pl.BlockSpec((_BM, DPAD), lambda i: (i, 0)),
      out_shape=jax.ShapeDtypeStruct((N_NODES, DPAD), jnp.float32),
  )(p, p, W1, W2p)


def _fin_body(r0_ref, r1_ref, b2_ref, o_ref):
  y = r0_ref[...] + r1_ref[...] + b2_ref[...]
  col = lax.broadcasted_iota(jnp.int32, y.shape, 1)
  ym = jnp.where(col < NCLASS, y, -jnp.inf)
  m = jnp.max(ym, axis=1, keepdims=True)
  lse = jnp.log(jnp.sum(jnp.exp(ym - m), axis=1, keepdims=True)) + m
  o_ref[...] = (y - lse)[:, :NCLASS]


def _fin(r, b2p):
  return pl.pallas_call(
      _fin_body,
      grid=(_NB,),
      in_specs=[
          pl.BlockSpec((_BM, DPAD), lambda i: (i, 0)),
          pl.BlockSpec((_BM, DPAD), lambda i: (i + _NB, 0)),
          pl.BlockSpec((1, DPAD), lambda i: (0, 0)),
      ],
      out_specs=pl.BlockSpec((_BM, NCLASS), lambda i: (i, 0)),
      out_shape=jax.ShapeDtypeStruct((N_NODES, NCLASS), jnp.float32),
  )(r, r, b2p)


def kernel(features, edge_index, W1, W2, b2):
  ei = edge_index.reshape(2, N_CHUNKS, CHUNK)
  p = _spmm128(features, ei)
  W2p = jnp.pad(W2, ((0, 0), (0, DPAD - NCLASS)))
  g = _mid(p, W1, W2p)
  r = _spmm64(g, ei)
  b2p = jnp.pad(b2, (0, DPAD - NCLASS)).reshape(1, DPAD)
  return _fin(r, b2p)


# speedup vs baseline: 1.1424x; 1.1102x over previous
import functools

import jax
import jax.numpy as jnp
from jax import lax
from jax.experimental import pallas as pl
from jax.experimental.pallas import tpu as pltpu
from jax.experimental.pallas import tpu_sc as plsc

N_NODES = 10000
N_EDGES = 320000
NFEAT = 128
NCLASS = 40
DPAD = 64

NC = 2
NS = 16
CHUNK = 125
EDGES_PER_TILE = N_EDGES // (NC * NS)
N_CHUNKS = EDGES_PER_TILE // CHUNK
HALF = N_CHUNKS // 2
ROWS_PER_TILE = N_NODES // NS
ZCOPIES = ROWS_PER_TILE // CHUNK


def _make_spmm(D):
  mesh = plsc.VectorSubcoreMesh(core_axis_name="c", subcore_axis_name="s")
  params = pltpu.CompilerParams(use_tc_tiling_on_sc=False)

  @functools.partial(
      pl.kernel,
      out_type=jax.ShapeDtypeStruct((NC * N_NODES, D), jnp.float32),
      compiler_params=params,
      mesh=mesh,
      scratch_types=[
          pltpu.VMEM((HALF, CHUNK), jnp.int32),
          pltpu.VMEM((HALF, CHUNK), jnp.int32),
          pltpu.VMEM((CHUNK, D), jnp.float32),
          pltpu.VMEM((CHUNK, D), jnp.float32),
          pltpu.VMEM_SHARED((N_NODES, D), jnp.float32),
          pltpu.SemaphoreType.DMA,
          pltpu.SemaphoreType.DMA,
      ],
  )
  def spmm(table_hbm, ei_hbm, out_hbm, sidx, didx, r0, r1, acc, sem0, sem1):
    c = lax.axis_index("c")
    s = lax.axis_index("s")
    wid = c * NS + s

    zero16 = jnp.zeros((16,), jnp.float32)

    def zrow(i, carry):
      for j in range(D // 16):
        r0[i, pl.ds(j * 16, 16)] = zero16
      return carry

    lax.fori_loop(0, CHUNK, zrow, 0)
    for k in range(ZCOPIES):
      pltpu.sync_copy(r0, acc.at[pl.ds((s * ZCOPIES + k) * CHUNK, CHUNK)])

    def wait0():
      pltpu.make_async_copy(table_hbm.at[sidx.at[0]], r0, sem0).wait()

    def wait1():
      pltpu.make_async_copy(table_hbm.at[sidx.at[0]], r1, sem1).wait()

    for h in range(N_CHUNKS // HALF):
      base = wid * N_CHUNKS + h * HALF
      pltpu.sync_copy(ei_hbm.at[0, pl.ds(base, HALF)], sidx)
      pltpu.sync_copy(ei_hbm.at[1, pl.ds(base, HALF)], didx)
      pltpu.async_copy(table_hbm.at[sidx.at[0]], r0, sem0)
      pltpu.async_copy(table_hbm.at[sidx.at[1]], r1, sem1)
      if h == 0:
        plsc.subcore_barrier()

      def body(i, carry):
        j0 = 2 * i
        wait0()
        pltpu.sync_copy(r0, acc.at[didx.at[j0]], add=True)
        pltpu.async_copy(
            table_hbm.at[sidx.at[jnp.minimum(j0 + 2, HALF - 1)]], r0, sem0)
        wait1()
        pltpu.sync_copy(r1, acc.at[didx.at[j0 + 1]], add=True)
        pltpu.async_copy(
            table_hbm.at[sidx.at[jnp.minimum(j0 + 3, HALF - 1)]], r1, sem1)
        return carry

      lax.fori_loop(0, HALF // 2, body, 0)
      wait0()
      wait1()

    plsc.subcore_barrier()

    pltpu.sync_copy(
        acc.at[pl.ds(s * ROWS_PER_TILE, ROWS_PER_TILE)],
        out_hbm.at[pl.ds(c * N_NODES + s * ROWS_PER_TILE, ROWS_PER_TILE)])

  return spmm


def _make_spmm_ring4(D):
  mesh = plsc.VectorSubcoreMesh(core_axis_name="c", subcore_axis_name="s")
  params = pltpu.CompilerParams(use_tc_tiling_on_sc=False)

  @functools.partial(
      pl.kernel,
      out_type=jax.ShapeDtypeStruct((NC * N_NODES, D), jnp.float32),
      compiler_params=params,
      mesh=mesh,
      scratch_types=[
          pltpu.VMEM((N_CHUNKS, CHUNK), jnp.int32),
          pltpu.VMEM((N_CHUNKS, CHUNK), jnp.int32),
          [pltpu.VMEM((CHUNK, D), jnp.float32) for _ in range(4)],
          [pltpu.SemaphoreType.DMA for _ in range(4)],
          pltpu.VMEM_SHARED((N_NODES, D), jnp.float32),
      ],
  )
  def spmm(table_hbm, ei_hbm, out_hbm, sidx, didx, bufs, sems, acc):
    c = lax.axis_index("c")
    s = lax.axis_index("s")
    wid = c * NS + s

    zero16 = jnp.zeros((16,), jnp.float32)
    r0 = bufs[0]

    def zrow(i, carry):
      for j in range(D // 16):
        r0[i, pl.ds(j * 16, 16)] = zero16
      return carry

    lax.fori_loop(0, CHUNK, zrow, 0)
    for k in range(ZCOPIES):
      pltpu.sync_copy(r0, acc.at[pl.ds((s * ZCOPIES + k) * CHUNK, CHUNK)])

    base = wid * N_CHUNKS
    pltpu.sync_copy(ei_hbm.at[0, pl.ds(base, N_CHUNKS)], sidx)
    pltpu.sync_copy(ei_hbm.at[1, pl.ds(base, N_CHUNKS)], didx)
    for k in range(4):
      pltpu.async_copy(table_hbm.at[sidx.at[k]], bufs[k], sems[k])
    plsc.subcore_barrier()

    def waitk(k):
      pltpu.make_async_copy(table_hbm.at[sidx.at[0]], bufs[k], sems[k]).wait()

    def body(i, carry):
      j0 = 4 * i
      for k in range(4):
        waitk(k)
        pltpu.sync_copy(bufs[k], acc.at[didx.at[j0 + k]], add=True)
        pltpu.async_copy(
            table_hbm.at[sidx.at[jnp.minimum(j0 + 4 + k, N_CHUNKS - 1)]],
            bufs[k], sems[k])
      return carry

    lax.fori_loop(0, N_CHUNKS // 4, body, 0)
    for k in range(4):
      waitk(k)

    plsc.subcore_barrier()

    pltpu.sync_copy(
        acc.at[pl.ds(s * ROWS_PER_TILE, ROWS_PER_TILE)],
        out_hbm.at[pl.ds(c * N_NODES + s * ROWS_PER_TILE, ROWS_PER_TILE)])

  return spmm


_spmm128 = _make_spmm(NFEAT)
_spmm64 = _make_spmm_ring4(DPAD)


_BM = 2000
_NB = N_NODES // _BM


def _mid_body(q0_ref, q1_ref, w1_ref, w2_ref, g_ref):
  x = q0_ref[...] + q1_ref[...]
  h = jnp.maximum(
      jnp.dot(x, w1_ref[...], preferred_element_type=jnp.float32), 0.0)
  g_ref[...] = jnp.dot(h, w2_ref[...], preferred_element_type=jnp.float32)


def _mid(p, W1, W2p):
  return pl.pallas_call(
      _mid_body,
      grid=(_NB,),
      in_specs=[
          pl.BlockSpec((_BM, NFEAT), lambda i: (i, 0)),
          pl.BlockSpec((_BM, NFEAT), lambda i: (i + _NB, 0)),
          pl.BlockSpec((NFEAT, NFEAT), lambda i: (0, 0)),
          pl.BlockSpec((NFEAT, DPAD), lambda i: (0, 0)),
      ],
      out_specs=pl.BlockSpec((_BM, DPAD), lambda i: (i, 0)),
      out_shape=jax.ShapeDtypeStruct((N_NODES, DPAD), jnp.float32),
  )(p, p, W1, W2p)


def _fin_body(r0_ref, r1_ref, b2_ref, o_ref):
  y = r0_ref[...] + r1_ref[...] + b2_ref[...]
  col = lax.broadcasted_iota(jnp.int32, y.shape, 1)
  ym = jnp.where(col < NCLASS, y, -jnp.inf)
  m = jnp.max(ym, axis=1, keepdims=True)
  lse = jnp.log(jnp.sum(jnp.exp(ym - m), axis=1, keepdims=True)) + m
  o_ref[...] = (y - lse)[:, :NCLASS]


def _fin(r, b2p):
  return pl.pallas_call(
      _fin_body,
      grid=(_NB,),
      in_specs=[
          pl.BlockSpec((_BM, DPAD), lambda i: (i, 0)),
          pl.BlockSpec((_BM, DPAD), lambda i: (i + _NB, 0)),
          pl.BlockSpec((1, DPAD), lambda i: (0, 0)),
      ],
      out_specs=pl.BlockSpec((_BM, NCLASS), lambda i: (i, 0)),
      out_shape=jax.ShapeDtypeStruct((N_NODES, NCLASS), jnp.float32),
  )(r, r, b2p)


def kernel(features, edge_index, W1, W2, b2):
  ei = edge_index.reshape(2, N_EDGES // CHUNK, CHUNK)
  p = _spmm128(features, ei)
  W2p = jnp.pad(W2, ((0, 0), (0, DPAD - NCLASS)))
  g = _mid(p, W1, W2p)
  r = _spmm64(g, ei)
  b2p = jnp.pad(b2, (0, DPAD - NCLASS)).reshape(1, DPAD)
  return _fin(r, b2p)
